# initial kernel scaffold (unmeasured)
import jax
import jax.numpy as jnp
from jax import lax
from jax.experimental import pallas as pl
from jax.experimental.pallas import tpu as pltpu

N_DEV = 32


def kernel(x, w_mat):
    x = x.astype(jnp.bfloat16)
    w = w_mat.astype(jnp.bfloat16)
    m, k = x.shape
    _, n = w.shape
    R = m // N_DEV
    H = n // 2

    def body(x_ref, w_ref, out_ref,
             acc_cw, acc_ccw, recv_cw, recv_ccw,
             ag_cw, ag_ccw, stage_cw, stage_ccw,
             send_sems, recv_sems, store_sems,
             rs_credit, ag_credit):
        me = lax.axis_index("i")
        left = lax.rem(me - 1 + N_DEV, N_DEV)
        right = lax.rem(me + 1, N_DEV)

        barrier_sem = pltpu.get_barrier_semaphore()
        for nbr in (left, right):
            pl.semaphore_signal(barrier_sem, inc=1, device_id=(nbr,),
                                device_id_type=pl.DeviceIdType.MESH)
        pl.semaphore_wait(barrier_sem, 2)

        def mm(c, col0):
            xa = x_ref[pl.ds(c * R, R), :]
            wb = w_ref[:, col0:col0 + H]
            return jnp.dot(xa, wb, preferred_element_type=jnp.float32)

        acc_cw[...] = mm(me, 0).astype(jnp.bfloat16)
        acc_ccw[...] = mm(me, H).astype(jnp.bfloat16)

        def rs_step(s, carry):
            @pl.when(s > 0)
            def _():
                pl.semaphore_wait(rs_credit.at[0], 1)
                pl.semaphore_wait(rs_credit.at[1], 1)

            cw = pltpu.make_async_remote_copy(
                src_ref=acc_cw, dst_ref=recv_cw,
                send_sem=send_sems.at[0], recv_sem=recv_sems.at[0],
                device_id=(right,), device_id_type=pl.DeviceIdType.MESH)
            ccw = pltpu.make_async_remote_copy(
                src_ref=acc_ccw, dst_ref=recv_ccw,
                send_sem=send_sems.at[1], recv_sem=recv_sems.at[1],
                device_id=(left,), device_id_type=pl.DeviceIdType.MESH)
            cw.start()
            ccw.start()
            cw.wait()
            ccw.wait()

            c1 = lax.rem(me - s - 1 + N_DEV, N_DEV)
            acc_cw[...] = (recv_cw[...].astype(jnp.float32)
                           + mm(c1, 0)).astype(jnp.bfloat16)
            c2 = lax.rem(me + s + 1, N_DEV)
            acc_ccw[...] = (recv_ccw[...].astype(jnp.float32)
                            + mm(c2, H)).astype(jnp.bfloat16)

            @pl.when(s < N_DEV - 2)
            def _():
                pl.semaphore_signal(rs_credit.at[0], inc=1, device_id=(left,),
                                    device_id_type=pl.DeviceIdType.MESH)
                pl.semaphore_signal(rs_credit.at[1], inc=1, device_id=(right,),
                                    device_id_type=pl.DeviceIdType.MESH)
            return carry

        lax.fori_loop(0, N_DEV - 1, rs_step, 0)

        def silu(y):
            return y * (1.0 / (1.0 + jnp.exp(-y)))

        z_cw = silu(acc_cw[...].astype(jnp.float32))
        ag_cw[0] = z_cw.astype(jnp.bfloat16)
        stage_cw[...] = z_cw
        mine_cw = lax.rem(me + 1, N_DEV)
        st = pltpu.make_async_copy(
            stage_cw, out_ref.at[pl.ds(mine_cw * R, R), pl.ds(0, H)],
            store_sems.at[0])
        st.start()
        st.wait()

        z_ccw = silu(acc_ccw[...].astype(jnp.float32))
        ag_ccw[0] = z_ccw.astype(jnp.bfloat16)
        stage_ccw[...] = z_ccw
        mine_ccw = lax.rem(me - 1 + N_DEV, N_DEV)
        st = pltpu.make_async_copy(
            stage_ccw, out_ref.at[pl.ds(mine_ccw * R, R), pl.ds(H, H)],
            store_sems.at[1])
        st.start()
        st.wait()

        def ag_step(t, carry):
            slot = lax.rem(t, 2)
            nslot = lax.rem(t + 1, 2)

            @pl.when(t > 0)
            def _():
                pl.semaphore_wait(ag_credit.at[0], 1)
                pl.semaphore_wait(ag_credit.at[1], 1)

            cw = pltpu.make_async_remote_copy(
                src_ref=ag_cw.at[slot], dst_ref=ag_cw.at[nslot],
                send_sem=send_sems.at[0], recv_sem=recv_sems.at[0],
                device_id=(right,), device_id_type=pl.DeviceIdType.MESH)
            ccw = pltpu.make_async_remote_copy(
                src_ref=ag_ccw.at[slot], dst_ref=ag_ccw.at[nslot],
                send_sem=send_sems.at[1], recv_sem=recv_sems.at[1],
                device_id=(left,), device_id_type=pl.DeviceIdType.MESH)
            cw.start()
            ccw.start()
            cw.wait()
            ccw.wait()

            c1 = lax.rem(me - t + N_DEV, N_DEV)
            stage_cw[...] = ag_cw[nslot].astype(jnp.float32)
            st1 = pltpu.make_async_copy(
                stage_cw, out_ref.at[pl.ds(c1 * R, R), pl.ds(0, H)],
                store_sems.at[0])
            st1.start()

            c2 = lax.rem(me + t, N_DEV)
            stage_ccw[...] = ag_ccw[nslot].astype(jnp.float32)
            st2 = pltpu.make_async_copy(
                stage_ccw, out_ref.at[pl.ds(c2 * R, R), pl.ds(H, H)],
                store_sems.at[1])
            st2.start()
            st1.wait()
            st2.wait()

            @pl.when(t < N_DEV - 2)
            def _():
                pl.semaphore_signal(ag_credit.at[0], inc=1, device_id=(left,),
                                    device_id_type=pl.DeviceIdType.MESH)
                pl.semaphore_signal(ag_credit.at[1], inc=1, device_id=(right,),
                                    device_id_type=pl.DeviceIdType.MESH)
            return carry

        lax.fori_loop(0, N_DEV - 1, ag_step, 0)

    return pl.pallas_call(
        body,
        out_shape=jax.ShapeDtypeStruct((m, n), jnp.float32),
        in_specs=[
            pl.BlockSpec(memory_space=pltpu.VMEM),
            pl.BlockSpec(memory_space=pltpu.VMEM),
        ],
        out_specs=pl.BlockSpec(memory_space=pltpu.ANY),
        scratch_shapes=[
            pltpu.VMEM((R, H), jnp.bfloat16),
            pltpu.VMEM((R, H), jnp.bfloat16),
            pltpu.VMEM((R, H), jnp.bfloat16),
            pltpu.VMEM((R, H), jnp.bfloat16),
            pltpu.VMEM((2, R, H), jnp.bfloat16),
            pltpu.VMEM((2, R, H), jnp.bfloat16),
            pltpu.VMEM((R, H), jnp.float32),
            pltpu.VMEM((R, H), jnp.float32),
            pltpu.SemaphoreType.DMA((2,)),
            pltpu.SemaphoreType.DMA((2,)),
            pltpu.SemaphoreType.DMA((2,)),
            pltpu.SemaphoreType.REGULAR((2,)),
            pltpu.SemaphoreType.REGULAR((2,)),
        ],
        compiler_params=pltpu.CompilerParams(collective_id=0),
    )(x, w)


# baseline (device time: 1730068 ns/iter reference)
import jax
import jax.numpy as jnp
from jax import lax
from jax.experimental import pallas as pl
from jax.experimental.pallas import tpu as pltpu

N_DEV = 32


def kernel(x, w_mat):
    x = x.astype(jnp.bfloat16)
    w = w_mat.astype(jnp.bfloat16)
    m, k = x.shape
    _, n = w.shape
    R = m // N_DEV
    H = n // 2

    def body(x_ref, w_ref, out_ref,
             acc_cw, acc_ccw, recv_cw, recv_ccw,
             ag_cw, ag_ccw, stage_cw, stage_ccw,
             send_sems, recv_sems, store_sems,
             rs_credit, ag_credit):
        me = lax.axis_index("i")
        left = lax.rem(me - 1 + N_DEV, N_DEV)
        right = lax.rem(me + 1, N_DEV)

        barrier_sem = pltpu.get_barrier_semaphore()
        for nbr in (left, right):
            pl.semaphore_signal(barrier_sem, inc=1, device_id=(nbr,),
                                device_id_type=pl.DeviceIdType.MESH)
        pl.semaphore_wait(barrier_sem, 2)

        def mm(c, col0):
            xa = x_ref[pl.ds(c * R, R), :]
            wb = w_ref[:, col0:col0 + H]
            return jnp.dot(xa, wb, preferred_element_type=jnp.float32)

        acc_cw[...] = mm(me, 0).astype(jnp.bfloat16)
        acc_ccw[...] = mm(me, H).astype(jnp.bfloat16)

        def rs_step(s, carry):
            @pl.when(s > 0)
            def _():
                pl.semaphore_wait(rs_credit.at[0], 1)
                pl.semaphore_wait(rs_credit.at[1], 1)

            cw = pltpu.make_async_remote_copy(
                src_ref=acc_cw, dst_ref=recv_cw,
                send_sem=send_sems.at[0], recv_sem=recv_sems.at[0],
                device_id=(right,), device_id_type=pl.DeviceIdType.MESH)
            ccw = pltpu.make_async_remote_copy(
                src_ref=acc_ccw, dst_ref=recv_ccw,
                send_sem=send_sems.at[1], recv_sem=recv_sems.at[1],
                device_id=(left,), device_id_type=pl.DeviceIdType.MESH)
            cw.start()
            ccw.start()
            cw.wait()
            ccw.wait()

            c1 = lax.rem(me - s - 1 + N_DEV, N_DEV)
            acc_cw[...] = (recv_cw[...].astype(jnp.float32)
                           + mm(c1, 0)).astype(jnp.bfloat16)
            c2 = lax.rem(me + s + 1, N_DEV)
            acc_ccw[...] = (recv_ccw[...].astype(jnp.float32)
                            + mm(c2, H)).astype(jnp.bfloat16)

            @pl.when(s < N_DEV - 2)
            def _():
                pl.semaphore_signal(rs_credit.at[0], inc=1, device_id=(left,),
                                    device_id_type=pl.DeviceIdType.MESH)
                pl.semaphore_signal(rs_credit.at[1], inc=1, device_id=(right,),
                                    device_id_type=pl.DeviceIdType.MESH)
            return carry

        lax.fori_loop(0, N_DEV - 1, rs_step, 0)

        def silu(y):
            return y * (1.0 / (1.0 + jnp.exp(-y)))

        z_cw = silu(acc_cw[...].astype(jnp.float32))
        ag_cw[0] = z_cw.astype(jnp.bfloat16)
        stage_cw[...] = z_cw
        mine_cw = lax.rem(me + 1, N_DEV)
        st = pltpu.make_async_copy(
            stage_cw, out_ref.at[pl.ds(mine_cw * R, R), pl.ds(0, H)],
            store_sems.at[0])
        st.start()
        st.wait()

        z_ccw = silu(acc_ccw[...].astype(jnp.float32))
        ag_ccw[0] = z_ccw.astype(jnp.bfloat16)
        stage_ccw[...] = z_ccw
        mine_ccw = lax.rem(me - 1 + N_DEV, N_DEV)
        st = pltpu.make_async_copy(
            stage_ccw, out_ref.at[pl.ds(mine_ccw * R, R), pl.ds(H, H)],
            store_sems.at[1])
        st.start()
        st.wait()

        def ag_step(t, carry):
            slot = lax.rem(t, 2)
            nslot = lax.rem(t + 1, 2)

            @pl.when(t > 0)
            def _():
                pl.semaphore_wait(ag_credit.at[0], 1)
                pl.semaphore_wait(ag_credit.at[1], 1)

            cw = pltpu.make_async_remote_copy(
                src_ref=ag_cw.at[slot], dst_ref=ag_cw.at[nslot],
                send_sem=send_sems.at[0], recv_sem=recv_sems.at[0],
                device_id=(right,), device_id_type=pl.DeviceIdType.MESH)
            ccw = pltpu.make_async_remote_copy(
                src_ref=ag_ccw.at[slot], dst_ref=ag_ccw.at[nslot],
                send_sem=send_sems.at[1], recv_sem=recv_sems.at[1],
                device_id=(left,), device_id_type=pl.DeviceIdType.MESH)
            cw.start()
            ccw.start()
            cw.wait()
            ccw.wait()

            c1 = lax.rem(me - t + N_DEV, N_DEV)
            stage_cw[...] = ag_cw[nslot].astype(jnp.float32)
            st1 = pltpu.make_async_copy(
                stage_cw, out_ref.at[pl.ds(c1 * R, R), pl.ds(0, H)],
                store_sems.at[0])
            st1.start()

            c2 = lax.rem(me + t, N_DEV)
            stage_ccw[...] = ag_ccw[nslot].astype(jnp.float32)
            st2 = pltpu.make_async_copy(
                stage_ccw, out_ref.at[pl.ds(c2 * R, R), pl.ds(H, H)],
                store_sems.at[1])
            st2.start()
            st1.wait()
            st2.wait()

            @pl.when(t < N_DEV - 2)
            def _():
                pl.semaphore_signal(ag_credit.at[0], inc=1, device_id=(left,),
                                    device_id_type=pl.DeviceIdType.MESH)
                pl.semaphore_signal(ag_credit.at[1], inc=1, device_id=(right,),
                                    device_id_type=pl.DeviceIdType.MESH)
            return carry

        lax.fori_loop(0, N_DEV - 1, ag_step, 0)

    return pl.pallas_call(
        body,
        out_shape=jax.ShapeDtypeStruct((m, n), jnp.float32),
        in_specs=[
            pl.BlockSpec(memory_space=pltpu.VMEM),
            pl.BlockSpec(memory_space=pltpu.VMEM),
        ],
        out_specs=pl.BlockSpec(memory_space=pl.ANY),
        scratch_shapes=[
            pltpu.VMEM((R, H), jnp.bfloat16),
            pltpu.VMEM((R, H), jnp.bfloat16),
            pltpu.VMEM((R, H), jnp.bfloat16),
            pltpu.VMEM((R, H), jnp.bfloat16),
            pltpu.VMEM((2, R, H), jnp.bfloat16),
            pltpu.VMEM((2, R, H), jnp.bfloat16),
            pltpu.VMEM((R, H), jnp.float32),
            pltpu.VMEM((R, H), jnp.float32),
            pltpu.SemaphoreType.DMA((2,)),
            pltpu.SemaphoreType.DMA((2,)),
            pltpu.SemaphoreType.DMA((2,)),
            pltpu.SemaphoreType.REGULAR((2,)),
            pltpu.SemaphoreType.REGULAR((2,)),
        ],
        compiler_params=pltpu.CompilerParams(collective_id=0),
    )(x, w)


# device time: 1666192 ns/iter; 1.0383x vs baseline; 1.0383x over previous
import jax
import jax.numpy as jnp
from jax import lax
from jax.experimental import pallas as pl
from jax.experimental.pallas import tpu as pltpu

N_DEV = 32
N_LANES = 4
MESH = pl.DeviceIdType.MESH


def kernel(x, w_mat):
    x = x.astype(jnp.bfloat16)
    w = w_mat.astype(jnp.bfloat16)
    m, k = x.shape
    _, n = w.shape
    R = m // N_DEV
    Q = n // N_LANES

    def body(x_ref, w_ref, out_ref,
             acc, recv, ag, stage,
             rs_send_sems, rs_recv_sems, ag_send_sems, ag_recv_sems,
             store_sems, rs_credit, ag_credit):
        me = lax.axis_index("i")
        left = lax.rem(me - 1 + N_DEV, N_DEV)
        right = lax.rem(me + 1, N_DEV)

        barrier_sem = pltpu.get_barrier_semaphore()
        for nbr in (left, right):
            pl.semaphore_signal(barrier_sem, inc=1, device_id=(nbr,),
                                device_id_type=MESH)
        pl.semaphore_wait(barrier_sem, 2)

        lane_col = [0 * Q, 1 * Q, 2 * Q, 3 * Q]

        def lane_dst(l):
            return right if l < 2 else left

        def lane_src(l):
            return left if l < 2 else right

        def rs_rdma(l, slot):
            return pltpu.make_async_remote_copy(
                src_ref=acc.at[l, slot], dst_ref=recv.at[l, slot],
                send_sem=rs_send_sems.at[l, slot],
                recv_sem=rs_recv_sems.at[l, slot],
                device_id=(lane_dst(l),), device_id_type=MESH)

        def ag_rdma(l, sslot, rslot):
            return pltpu.make_async_remote_copy(
                src_ref=ag.at[l, sslot], dst_ref=ag.at[l, rslot],
                send_sem=ag_send_sems.at[l, sslot],
                recv_sem=ag_recv_sems.at[l, rslot],
                device_id=(lane_dst(l),), device_id_type=MESH)

        def mm(c, col0):
            xa = x_ref[pl.ds(c * R, R), :]
            wb = w_ref[:, col0:col0 + Q]
            return jnp.dot(xa, wb, preferred_element_type=jnp.float32)

        for l in range(N_LANES):
            acc[l, 0] = mm(me, lane_col[l]).astype(jnp.bfloat16)

        def rs_step(s, carry):
            slot = lax.rem(s, 2)
            nslot = lax.rem(s + 1, 2)

            @pl.when(s >= 2)
            def _():
                for l in range(N_LANES):
                    pl.semaphore_wait(rs_credit.at[l], 1)

            for l in range(N_LANES):
                rs_rdma(l, slot).start()

            c_cw = lax.rem(me - s - 1 + N_DEV, N_DEV)
            c_ccw = lax.rem(me + s + 1, N_DEV)
            mms = [mm(c_cw if l < 2 else c_ccw, lane_col[l])
                   for l in range(N_LANES)]

            for l in range(N_LANES):
                rs_rdma(l, slot).wait_recv()
                @pl.when(s >= 1)
                def _(l=l, nslot=nslot):
                    rs_rdma(l, nslot).wait_send()
                acc[l, nslot] = (recv[l, slot].astype(jnp.float32)
                                 + mms[l]).astype(jnp.bfloat16)

                @pl.when(s <= N_DEV - 4)
                def _(l=l):
                    pl.semaphore_signal(rs_credit.at[l], inc=1,
                                        device_id=(lane_src(l),),
                                        device_id_type=MESH)
            return carry

        lax.fori_loop(0, N_DEV - 1, rs_step, 0)

        for l in range(N_LANES):
            rs_rdma(l, 0).wait_send()

        def silu(y):
            return y * (1.0 / (1.0 + jnp.exp(-y)))

        mine_cw = lax.rem(me + 1, N_DEV)
        mine_ccw = lax.rem(me - 1 + N_DEV, N_DEV)
        for l in range(N_LANES):
            z = silu(acc[l, 1].astype(jnp.float32))
            ag[l, 0] = z.astype(jnp.bfloat16)
            stage[l, 0] = z
            c_own = mine_cw if l < 2 else mine_ccw
            pltpu.make_async_copy(
                stage.at[l, 0],
                out_ref.at[pl.ds(c_own * R, R), pl.ds(lane_col[l], Q)],
                store_sems.at[l, 0]).start()

        def ag_step(t, carry):
            sslot = lax.rem(t, 3)
            rslot = lax.rem(t + 1, 3)
            stg = lax.rem(t, 2)

            @pl.when(t >= 3)
            def _():
                for l in range(N_LANES):
                    pl.semaphore_wait(ag_credit.at[l], 1)

            for l in range(N_LANES):
                ag_rdma(l, sslot, rslot).start()

            c_cw = lax.rem(me - t + N_DEV, N_DEV)
            c_ccw = lax.rem(me + t, N_DEV)
            for l in range(N_LANES):
                ag_rdma(l, sslot, rslot).wait_recv()

                @pl.when((t >= 2) | (t == 0))
                def _(l=l, stg=stg):
                    pltpu.make_async_copy(stage.at[l, stg], stage.at[l, stg],
                                          store_sems.at[l, stg]).wait()

                c = c_cw if l < 2 else c_ccw
                stage[l, stg] = ag[l, rslot].astype(jnp.float32)
                pltpu.make_async_copy(
                    stage.at[l, stg],
                    out_ref.at[pl.ds(c * R, R), pl.ds(lane_col[l], Q)],
                    store_sems.at[l, stg]).start()

                @pl.when(t >= 1)
                def _(l=l, t=t):
                    prev = lax.rem(t + 2, 3)
                    ag_rdma(l, prev, lax.rem(t, 3)).wait_send()
                @pl.when((t >= 2) & (t <= N_DEV - 3))
                def _(l=l):
                    pl.semaphore_signal(ag_credit.at[l], inc=1,
                                        device_id=(lane_src(l),),
                                        device_id_type=MESH)
            return carry

        lax.fori_loop(0, N_DEV - 1, ag_step, 0)

        for l in range(N_LANES):
            ag_rdma(l, 0, 1).wait_send()
            for sslot in range(2):
                pltpu.make_async_copy(stage.at[l, sslot], stage.at[l, sslot],
                                      store_sems.at[l, sslot]).wait()

    return pl.pallas_call(
        body,
        out_shape=jax.ShapeDtypeStruct((m, n), jnp.float32),
        in_specs=[
            pl.BlockSpec(memory_space=pltpu.VMEM),
            pl.BlockSpec(memory_space=pltpu.VMEM),
        ],
        out_specs=pl.BlockSpec(memory_space=pl.ANY),
        scratch_shapes=[
            pltpu.VMEM((N_LANES, 2, R, Q), jnp.bfloat16),
            pltpu.VMEM((N_LANES, 2, R, Q), jnp.bfloat16),
            pltpu.VMEM((N_LANES, 3, R, Q), jnp.bfloat16),
            pltpu.VMEM((N_LANES, 2, R, Q), jnp.float32),
            pltpu.SemaphoreType.DMA((N_LANES, 2)),
            pltpu.SemaphoreType.DMA((N_LANES, 2)),
            pltpu.SemaphoreType.DMA((N_LANES, 3)),
            pltpu.SemaphoreType.DMA((N_LANES, 3)),
            pltpu.SemaphoreType.DMA((N_LANES, 2)),
            pltpu.SemaphoreType.REGULAR((N_LANES,)),
            pltpu.SemaphoreType.REGULAR((N_LANES,)),
        ],
        compiler_params=pltpu.CompilerParams(collective_id=0),
    )(x, w)
